# baseline (speedup 1.0000x reference)
import functools

import jax
import jax.numpy as jnp
from jax.experimental import pallas as pl
from jax.experimental.pallas import tpu as pltpu

_BM = 600
_S = 2


def _gcn_stream_kernel(
    nfull, tail, x_ref, adj_hbm, w_ref, b_ref, o_ref, buf, sems
):
    x = x_ref[...]
    w = w_ref[...]
    b = b_ref[...]

    def start_copy(chunk, slot):
        pltpu.make_async_copy(
            adj_hbm.at[pl.ds(chunk * _BM, _BM), :],
            buf.at[slot],
            sems.at[slot],
        ).start()

    tail_slot = nfull % _S

    def start_tail_copy():
        pltpu.make_async_copy(
            adj_hbm.at[pl.ds(nfull * _BM, tail), :],
            buf.at[tail_slot, pl.ds(0, tail), :],
            sems.at[tail_slot],
        ).start()

    for s in range(min(_S, nfull)):
        start_copy(s, s)

    def step(g, carry):
        for s in range(_S):
            i = g * _S + s
            pltpu.make_async_copy(
                adj_hbm.at[pl.ds(i * _BM, _BM), :],
                buf.at[s],
                sems.at[s],
            ).wait()
            t = jnp.dot(buf[s], x, preferred_element_type=jnp.float32)
            o_ref[pl.ds(i * _BM, _BM), :] = (
                jnp.dot(t, w, preferred_element_type=jnp.float32) + b
            )

            nxt = i + _S

            @pl.when(nxt < nfull)
            def _():
                start_copy(nxt, s)

            if tail > 0:

                @pl.when(nxt == nfull)
                def _():
                    start_tail_copy()

        return carry

    assert nfull % _S == 0
    jax.lax.fori_loop(0, nfull // _S, step, 0)

    if tail > 0:
        pltpu.make_async_copy(
            adj_hbm.at[pl.ds(nfull * _BM, tail), :],
            buf.at[tail_slot, pl.ds(0, tail), :],
            sems.at[tail_slot],
        ).wait()
        tt = jnp.dot(
            buf[tail_slot, 0:tail, :], x, preferred_element_type=jnp.float32
        )
        o_ref[pl.ds(nfull * _BM, tail), :] = (
            jnp.dot(tt, w, preferred_element_type=jnp.float32) + b
        )


@jax.jit
def kernel(input, adj, W, b):
    n, d_in = input.shape
    d_out = W.shape[1]
    m = adj.shape[0]
    nfull = (m // _BM // _S) * _S
    tail = m - nfull * _BM
    assert tail % 8 == 0 and 0 <= tail <= _BM
    b2 = b.reshape(1, d_out)
    return pl.pallas_call(
        functools.partial(_gcn_stream_kernel, nfull, tail),
        in_specs=[
            pl.BlockSpec(memory_space=pltpu.VMEM),
            pl.BlockSpec(memory_space=pl.ANY),
            pl.BlockSpec(memory_space=pltpu.VMEM),
            pl.BlockSpec(memory_space=pltpu.VMEM),
        ],
        out_specs=pl.BlockSpec(memory_space=pltpu.VMEM),
        out_shape=jax.ShapeDtypeStruct((m, d_out), jnp.float32),
        scratch_shapes=[
            pltpu.VMEM((_S, _BM, n), jnp.float32),
            pltpu.SemaphoreType.DMA((_S,)),
        ],
        compiler_params=pltpu.CompilerParams(
            vmem_limit_bytes=64 * 1024 * 1024,
        ),
    )(input, adj, W, b2)


# speedup vs baseline: 1.0345x; 1.0345x over previous
import functools

import jax
import jax.numpy as jnp
from jax.experimental import pallas as pl
from jax.experimental.pallas import tpu as pltpu

_BM = 400


def _gcn_block_kernel(x_ref, adj_ref, w_ref, b_ref, o_ref):
    t = jnp.dot(adj_ref[...], x_ref[...], preferred_element_type=jnp.float32)
    o_ref[...] = (
        jnp.dot(t, w_ref[...], preferred_element_type=jnp.float32) + b_ref[...]
    )


@jax.jit
def kernel(input, adj, W, b):
    n, d_in = input.shape
    d_out = W.shape[1]
    b2 = b.reshape(1, d_out)
    grid = (pl.cdiv(adj.shape[0], _BM),)
    return pl.pallas_call(
        _gcn_block_kernel,
        grid=grid,
        in_specs=[
            pl.BlockSpec((n, d_in), lambda i: (0, 0)),
            pl.BlockSpec((_BM, n), lambda i: (i, 0)),
            pl.BlockSpec((d_in, d_out), lambda i: (0, 0)),
            pl.BlockSpec((1, d_out), lambda i: (0, 0)),
        ],
        out_specs=pl.BlockSpec((_BM, d_out), lambda i: (i, 0)),
        out_shape=jax.ShapeDtypeStruct((adj.shape[0], d_out), jnp.float32),
        compiler_params=pltpu.CompilerParams(
            dimension_semantics=("parallel",),
        ),
    )(input, adj, W, b2)
